# Initial kernel scaffold; baseline (speedup 1.0000x reference)
import jax
import jax.numpy as jnp
from jax.experimental import pallas as pl


def kernel(features, center, curvatures):
    raise NotImplementedError("write your pallas kernel here")


# speedup vs baseline: 8.5669x; 8.5669x over previous
import functools

import jax
import jax.numpy as jnp
from jax import lax
from jax.experimental import pallas as pl
from jax.experimental.pallas import tpu as pltpu
from jax.experimental.pallas import tpu_sc as plsc

_CTHR = 0.01
_K = 5


_QT = 256


def _top5_body(cenq_ref, cenk_ref, out_ref):
    qt = cenq_ref.shape[0]
    m = cenk_ref.shape[1]
    qx = cenq_ref[:, 0:1]
    qy = cenq_ref[:, 1:2]
    qz = cenq_ref[:, 2:3]
    kx = cenk_ref[0:1, :]
    ky = cenk_ref[1:2, :]
    kz = cenk_ref[2:3, :]
    sqq = qx * qx + qy * qy + qz * qz
    sqk = kx * kx + ky * ky + kz * kz
    dot = jax.lax.dot_general(
        cenq_ref[...], cenk_ref[...],
        dimension_numbers=(((1,), (0,)), ((), ())),
        precision=lax.Precision.DEFAULT,
        preferred_element_type=jnp.float32)
    d2 = jnp.maximum((sqq + sqk) - 2.0 * dot, 0.0)
    kiota = lax.broadcasted_iota(jnp.int32, (qt, m), 1)
    big = jnp.int32(2**30)
    inf = jnp.float32(jnp.inf)
    for j in range(_K):
        mn = jnp.min(d2, axis=1, keepdims=True)
        idxj = jnp.min(jnp.where(d2 == mn, kiota, big), axis=1)
        out_ref[0, :, j] = idxj
        if j + 1 < _K:
            d2 = jnp.where(kiota == idxj[:, None], inf, d2)
    zero = jnp.zeros((qt,), jnp.int32)
    for j in range(_K, 8):
        out_ref[0, :, j] = zero


def _top5(cenq, cenk):
    m = cenq.shape[0]
    grid = m // _QT
    out = pl.pallas_call(
        _top5_body,
        grid=(grid,),
        in_specs=[
            pl.BlockSpec((_QT, 8), lambda i: (i, 0)),
            pl.BlockSpec((8, m), lambda i: (0, 0)),
        ],
        out_specs=pl.BlockSpec((1, _QT, 8), lambda i: (i, 0, 0)),
        out_shape=jax.ShapeDtypeStruct((grid, _QT, 8), jnp.int32),
    )(cenq, cenk)
    return out.reshape(m, 8)


_PC = 64


def _rsqrt_nr(x):
    i = plsc.bitcast(x, jnp.int32)
    i = jnp.int32(0x5F3759DF) - (i >> 1)
    y = plsc.bitcast(i, jnp.float32)
    for _ in range(3):
        y = y * (1.5 - 0.5 * x * y * y)
    return y


def _sc_agg_make(m, c, nw):
    pw = m // nw
    nch = pw // _PC
    lanes = 16
    nk = c // lanes
    mesh = plsc.VectorSubcoreMesh(core_axis_name="c", subcore_axis_name="s",
                                  num_cores=2, num_subcores=16)

    @functools.partial(
        pl.kernel,
        out_type=jax.ShapeDtypeStruct((m, 3 * c), jnp.float32),
        mesh=mesh,
        compiler_params=pltpu.CompilerParams(needs_layout_passes=False),
        scratch_types=[
            pltpu.VMEM((_PC * _K,), jnp.int32),
            pltpu.VMEM((_PC,), jnp.float32),
            pltpu.VMEM((_PC * _K, c), jnp.float32),
            pltpu.VMEM((_PC, 3 * c), jnp.float32),
            pltpu.VMEM((3 * c,), jnp.float32),
            pltpu.SemaphoreType.DMA,
        ],
    )
    def sc_agg(f0, f1, f2, idx_hbm, curv_hbm, out_hbm,
               idx_v, curv_v, rows_v, out_v, flat_v, sem):
        wid = lax.axis_index("s") * 2 + lax.axis_index("c")
        lane = lax.iota(jnp.int32, lanes)
        pool = [[3 * (lanes * k + lane) + t for t in range(3)]
                for k in range(nk)]
        third = jnp.full((lanes,), 1.0 / 3.0, jnp.float32)

        def chunk_body(ch, carry):
            base = wid * pw + ch * _PC
            pltpu.sync_copy(idx_hbm.at[pl.ds(pl.multiple_of(base * _K, 8),
                                             _PC * _K)], idx_v)
            pltpu.sync_copy(curv_hbm.at[pl.ds(pl.multiple_of(base, 8), _PC)],
                            curv_v)
            for l, tbl in enumerate((f0, f1, f2)):
                cops = []
                for off in range(0, _PC * _K, 128):
                    sz = min(128, _PC * _K - off)
                    cops.append(pltpu.async_copy(
                        tbl.at[idx_v.at[pl.ds(off, sz)]],
                        rows_v.at[pl.ds(off, sz)], sem))
                for cp in cops:
                    cp.wait()

                def point_body(p, pcarry, l=l):
                    cv = plsc.load_gather(
                        curv_v, [jnp.full((lanes,), p, jnp.int32)])
                    masked = cv[0] < _CTHR

                    @pl.when(jnp.logical_not(masked))
                    def _light():
                        for k in range(nk):
                            out_v[p, pl.ds(l * c + lanes * k, lanes)] = (
                                rows_v[_K * p, pl.ds(lanes * k, lanes)])

                    @pl.when(masked)
                    def _heavy():
                        r0 = [rows_v[_K * p, pl.ds(lanes * k, lanes)]
                              for k in range(nk)]
                        acc = list(r0)
                        wsum = jnp.full((lanes,), 1.0, jnp.float32)
                        out3 = None
                        for j in range(1, _K):
                            rj = [rows_v[_K * p + j, pl.ds(lanes * k, lanes)]
                                  for k in range(nk)]
                            sq = jnp.zeros((lanes,), jnp.float32)
                            for k in range(nk):
                                d = rj[k] - r0[k]
                                sq = sq + d * d
                            sqn_v = sq
                            for s in (8, 4, 2, 1):
                                flat_v[pl.ds(0, lanes)] = sqn_v
                                sqn_v = sqn_v + plsc.load_gather(
                                    flat_v, [lane ^ s])
                            dist = jnp.where(sqn_v > 0.0,
                                             sqn_v * _rsqrt_nr(sqn_v), 0.0)
                            w = jnp.exp(-dist)
                            for k in range(nk):
                                acc[k] = acc[k] + w * rj[k]
                            wsum = wsum + w
                            if j == 2:
                                out3 = [acc[k] / wsum for k in range(nk)]
                        out5 = [acc[k] / wsum for k in range(nk)]
                        for k in range(nk):
                            flat_v[pl.ds(lanes * k, lanes)] = r0[k]
                            flat_v[pl.ds(c + lanes * k, lanes)] = out3[k]
                            flat_v[pl.ds(2 * c + lanes * k, lanes)] = out5[k]
                        for k in range(nk):
                            g = plsc.load_gather(flat_v, [pool[k][0]])
                            g = g + plsc.load_gather(flat_v, [pool[k][1]])
                            g = g + plsc.load_gather(flat_v, [pool[k][2]])
                            out_v[p, pl.ds(l * c + lanes * k, lanes)] = (
                                g * third)

                    return pcarry

                lax.fori_loop(0, _PC, point_body, 0)
            pltpu.sync_copy(out_v,
                            out_hbm.at[pl.ds(pl.multiple_of(base, 8), _PC)])
            return carry

        lax.fori_loop(0, nch, chunk_body, 0)

    return sc_agg


def kernel(features, center, curvatures):
    l, b, n, c = features.shape
    m = b * n
    feats = features.reshape(l, m, c)
    cen = jnp.transpose(center, (1, 0, 2)).reshape(3, m)
    cenk = jnp.concatenate([cen, jnp.zeros((5, m), jnp.float32)], axis=0)
    cenq = cenk.T
    idx8 = _top5(cenq, cenk)
    idxflat = idx8[:, :_K].reshape(m * _K)
    nw = 32
    out = _sc_agg_make(m, c, nw)(feats[0], feats[1], feats[2],
                                 idxflat, curvatures)
    return out.reshape(b, n, l, c).astype(jnp.float16)
